# baseline (speedup 1.0000x reference)
import functools

import jax
import jax.numpy as jnp
from jax import lax
from jax.experimental import pallas as pl
from jax.experimental.pallas import tpu as pltpu

_K = 1024
_M = 1024


def _vq_body(flat_ref, cb_ref, xsq_ref, cbsq_ref, idx_ref, q_ref):
    ft = flat_ref[...]
    cb = cb_ref[...]
    cross = lax.dot_general(ft, cb, (((1,), (1,)), ((), ())),
                            preferred_element_type=jnp.float32)
    t = (xsq_ref[...] + cbsq_ref[...]) - 2.0 * cross
    dist = jnp.sqrt(jnp.clip(t, 0.0, None))
    minv = jnp.min(dist, axis=1, keepdims=True)
    kiota = lax.broadcasted_iota(jnp.int32, (_M, _K), 1)
    masked = jnp.where(dist == minv, kiota, _K)
    idx = jnp.min(masked, axis=1, keepdims=True)
    idx_ref[...] = idx
    onehot = (kiota == idx).astype(jnp.float32)
    q_ref[...] = lax.dot_general(onehot, cb, (((1,), (0,)), ((), ())),
                                 preferred_element_type=jnp.float32)


def kernel(x, codebook):
    b, c, h, w = x.shape
    n = h * w
    flat = jnp.transpose(x, (0, 2, 3, 1)).reshape(b, n, c).astype(jnp.float32)
    x_sq = jnp.sum(flat * flat, axis=-1, keepdims=True)
    cb_sq = jnp.sum(codebook * codebook, axis=-1)
    rows = b * n
    flat2 = flat.reshape(rows, c)
    xsq2 = x_sq.reshape(rows, 1)
    cbsq2 = cb_sq.reshape(1, _K)
    grid = rows // _M
    idx, q = pl.pallas_call(
        _vq_body,
        grid=(grid,),
        in_specs=[
            pl.BlockSpec((_M, c), lambda i: (i, 0)),
            pl.BlockSpec((_K, c), lambda i: (0, 0)),
            pl.BlockSpec((_M, 1), lambda i: (i, 0)),
            pl.BlockSpec((1, _K), lambda i: (0, 0)),
        ],
        out_specs=[
            pl.BlockSpec((_M, 1), lambda i: (i, 0)),
            pl.BlockSpec((_M, c), lambda i: (i, 0)),
        ],
        out_shape=[
            jax.ShapeDtypeStruct((rows, 1), jnp.int32),
            jax.ShapeDtypeStruct((rows, c), jnp.float32),
        ],
    )(flat2, codebook, xsq2, cbsq2)
    quantized = jnp.transpose(q.reshape(b, h, w, c), (0, 3, 1, 2))
    embed_index = idx.reshape(b, h, w)
    loss = jnp.array([0.0], dtype=jnp.float32)
    return (quantized, embed_index, loss)
